# baseline (speedup 1.0000x reference)
import jax
import jax.numpy as jnp
from jax.experimental import pallas as pl
from jax.experimental.pallas import tpu as pltpu

_S = 64.0
_MARGIN = 0.5
_C2 = _S * 1.4426950408889634


def _body_factory(B, C, BR, BC):
    NC = pl.cdiv(C, BC)
    NSL = BC // 128

    def body(x_ref, lab_ref, out_ref, m_ref, s_ref, p_ref):
        j = pl.program_id(1)

        @pl.when(j == 0)
        def _init():
            m_ref[...] = jnp.full((BR, 1), -jnp.inf, jnp.float32)
            s_ref[...] = jnp.zeros((BR, 1), jnp.float32)
            p_ref[...] = jnp.zeros((BR, 128), jnp.float32)

        lab = lab_ref[...]

        def _slice(k, masked):
            xk = x_ref[:, k * 128:(k + 1) * 128]
            if masked:
                lanes = jax.lax.broadcasted_iota(jnp.int32, (BR, 128), 1)
                gcol = j * BC + k * 128 + lanes
                xk = jnp.where(gcol < C, xk, -jnp.inf)
            return xk

        def _block(masked):
            lanes = jax.lax.broadcasted_iota(jnp.int32, (BR, 128), 1)
            macc = jnp.full((BR, 128), -jnp.inf, jnp.float32)
            pacc = p_ref[...]
            for k in range(NSL):
                xk = _slice(k, masked)
                macc = jnp.maximum(macc, xk)
                pacc = pacc + jnp.where(lanes == lab - (j * BC + k * 128),
                                        xk, 0.0)
            p_ref[...] = pacc
            m_old = m_ref[...]
            m_new = jnp.maximum(m_old, jnp.max(macc, axis=1, keepdims=True))
            mc = m_new * _C2
            sacc = jnp.zeros((BR, 128), jnp.float32)
            for k in range(NSL):
                sacc = sacc + jnp.exp2(_slice(k, masked) * _C2 - mc)
            bs = jnp.sum(sacc, axis=1, keepdims=True)
            s_ref[...] = s_ref[...] * jnp.exp2(m_old * _C2 - mc) + bs
            m_ref[...] = m_new

        if C % BC != 0:
            @pl.when(j < NC - 1)
            def _fast():
                _block(False)

            @pl.when(j == NC - 1)
            def _slow():
                _block(True)
        else:
            _block(False)

        @pl.when(j == NC - 1)
        def _finish():
            m = m_ref[...]
            s = s_ref[...]
            a_x = jnp.sum(p_ref[...], axis=1, keepdims=True)
            q = jnp.exp(-_S * _MARGIN)
            ea = jnp.exp2(a_x * _C2 - m * _C2)
            s_adj = jnp.maximum(s - ea * (1.0 - q), ea * q)
            out_ref[...] = _S * m + jnp.log(s_adj) - _S * (a_x - _MARGIN)

    return body, NC


def _row_losses(cosine, lab2d, BR, BC):
    B, C = cosine.shape
    body, NC = _body_factory(B, C, BR, BC)
    return pl.pallas_call(
        body,
        grid=(B // BR, NC),
        in_specs=[
            pl.BlockSpec((BR, BC), lambda i, j: (i, j)),
            pl.BlockSpec((BR, 1), lambda i, j: (i, 0)),
        ],
        out_specs=pl.BlockSpec((BR, 1), lambda i, j: (i, 0)),
        out_shape=jax.ShapeDtypeStruct((B, 1), jnp.float32),
        scratch_shapes=[
            pltpu.VMEM((BR, 1), jnp.float32),
            pltpu.VMEM((BR, 1), jnp.float32),
            pltpu.VMEM((BR, 128), jnp.float32),
        ],
        compiler_params=pltpu.CompilerParams(
            dimension_semantics=("parallel", "arbitrary")
        ),
    )(cosine, lab2d)


def _mean_body(x_ref, out_ref):
    B = x_ref.shape[0]
    out_ref[...] = jnp.full((1, 1), jnp.sum(x_ref[...]) * (1.0 / B),
                            jnp.float32)


def _mean_call(row_losses):
    B = row_losses.shape[0]
    out = pl.pallas_call(
        _mean_body,
        out_shape=jax.ShapeDtypeStruct((1, 1), jnp.float32),
    )(row_losses)
    return out[0, 0]


@jax.jit
def kernel(cosine, label):
    B, _ = cosine.shape
    lab2d = label.astype(jnp.int32).reshape(B, 1)
    rl = _row_losses(cosine, lab2d, 256, 4096)
    return _mean_call(rl)


# speedup vs baseline: 2.3944x; 2.3944x over previous
import jax
import jax.numpy as jnp
from jax.experimental import pallas as pl
from jax.experimental.pallas import tpu as pltpu

_S = 64.0
_MARGIN = 0.5
_C2 = _S * 1.4426950408889634


def _body_factory(B, C, BCC):
    NCB = pl.cdiv(C, BCC)
    NSL = BCC // 8

    def body(x_ref, lab_ref, out_ref, m_ref, s_ref, p_ref):
        j = pl.program_id(0)

        @pl.when(j == 0)
        def _init():
            m_ref[...] = jnp.full((1, B), -jnp.inf, jnp.float32)
            s_ref[...] = jnp.zeros((1, B), jnp.float32)
            p_ref[...] = jnp.zeros((8, B), jnp.float32)

        lab = lab_ref[...]
        rows8 = jax.lax.broadcasted_iota(jnp.int32, (8, B), 0)

        def _slice(k, masked):
            xk = x_ref[k * 8:(k + 1) * 8, :]
            if masked:
                gid = j * BCC + k * 8 + rows8
                xk = jnp.where(gid < C, xk, -jnp.inf)
            return xk

        def _block(masked):
            macc = jnp.full((8, B), -jnp.inf, jnp.float32)
            pacc = p_ref[...]
            for k in range(NSL):
                xk = _slice(k, masked)
                macc = jnp.maximum(macc, xk)
                pacc = pacc + jnp.where(rows8 == lab - (j * BCC + k * 8),
                                        xk, 0.0)
            p_ref[...] = pacc
            m_old = m_ref[...]
            m_new = jnp.maximum(m_old, jnp.max(macc, axis=0, keepdims=True))
            mc = m_new * _C2
            sacc = jnp.zeros((8, B), jnp.float32)
            for k in range(NSL):
                sacc = sacc + jnp.exp2(_slice(k, masked) * _C2 - mc)
            bs = jnp.sum(sacc, axis=0, keepdims=True)
            s_ref[...] = s_ref[...] * jnp.exp2(m_old * _C2 - mc) + bs
            m_ref[...] = m_new

        if C % BCC != 0:
            @pl.when(j < NCB - 1)
            def _fast():
                _block(False)

            @pl.when(j == NCB - 1)
            def _slow():
                _block(True)
        else:
            _block(False)

        @pl.when(j == NCB - 1)
        def _finish():
            m = m_ref[...]
            s = s_ref[...]
            a_x = jnp.sum(p_ref[...], axis=0, keepdims=True)
            q = jnp.exp(-_S * _MARGIN)
            ea = jnp.exp2(a_x * _C2 - m * _C2)
            s_adj = jnp.maximum(s - ea * (1.0 - q), ea * q)
            out_ref[...] = _S * m + jnp.log(s_adj) - _S * (a_x - _MARGIN)

    return body, NCB


def _row_losses(cosT, lab2d, BCC):
    C, B = cosT.shape
    body, NCB = _body_factory(B, C, BCC)
    return pl.pallas_call(
        body,
        grid=(NCB,),
        in_specs=[
            pl.BlockSpec((BCC, B), lambda j: (j, 0)),
            pl.BlockSpec((1, B), lambda j: (0, 0)),
        ],
        out_specs=pl.BlockSpec((1, B), lambda j: (0, 0)),
        out_shape=jax.ShapeDtypeStruct((1, B), jnp.float32),
        scratch_shapes=[
            pltpu.VMEM((1, B), jnp.float32),
            pltpu.VMEM((1, B), jnp.float32),
            pltpu.VMEM((8, B), jnp.float32),
        ],
        compiler_params=pltpu.CompilerParams(
            dimension_semantics=("arbitrary",)
        ),
    )(cosT, lab2d)


def _mean_body(x_ref, out_ref):
    B = x_ref.shape[1]
    out_ref[...] = jnp.full((1, 1), jnp.sum(x_ref[...]) * (1.0 / B),
                            jnp.float32)


def _mean_call(row_losses):
    out = pl.pallas_call(
        _mean_body,
        out_shape=jax.ShapeDtypeStruct((1, 1), jnp.float32),
    )(row_losses)
    return out[0, 0]


@jax.jit
def kernel(cosine, label):
    B, _ = cosine.shape
    lab2d = label.astype(jnp.int32).reshape(1, B)
    rl = _row_losses(cosine.T, lab2d, 512)
    return _mean_call(rl)


# speedup vs baseline: 3.0383x; 1.2689x over previous
import jax
import jax.numpy as jnp
from jax.experimental import pallas as pl
from jax.experimental.pallas import tpu as pltpu

_S = 64.0
_MARGIN = 0.5
_C2 = _S * 1.4426950408889634


def _body_factory(B, C, BCC):
    NCB = pl.cdiv(C, BCC)
    NSL = BCC // 8

    def body(x_ref, lab_ref, out_ref, m_ref, s_ref, p_ref):
        j = pl.program_id(0)

        @pl.when(j == 0)
        def _init():
            m_ref[...] = jnp.full((1, B), -jnp.inf, jnp.float32)
            s_ref[...] = jnp.zeros((1, B), jnp.float32)
            p_ref[...] = jnp.zeros((8, B), jnp.float32)

        lab = lab_ref[...]
        rows8 = jax.lax.broadcasted_iota(jnp.int32, (8, B), 0)

        def _slice(k, masked):
            xk = x_ref[k * 8:(k + 1) * 8, :]
            if masked:
                gid = j * BCC + k * 8 + rows8
                xk = jnp.where(gid < C, xk, -jnp.inf)
            return xk

        def _block(masked):
            macc = jnp.full((8, B), -jnp.inf, jnp.float32)
            pacc = p_ref[...]
            for k in range(NSL):
                xk = _slice(k, masked)
                macc = jnp.maximum(macc, xk)
                pacc = pacc + jnp.where(rows8 == lab - (j * BCC + k * 8),
                                        xk, 0.0)
            p_ref[...] = pacc
            m_old = m_ref[...]
            m_new = jnp.maximum(m_old, jnp.max(macc, axis=0, keepdims=True))
            mc = m_new * _C2
            sacc = jnp.zeros((8, B), jnp.float32)
            for k in range(NSL):
                sacc = sacc + jnp.exp2(_slice(k, masked) * _C2 - mc)
            bs = jnp.sum(sacc, axis=0, keepdims=True)
            s_ref[...] = s_ref[...] * jnp.exp2(m_old * _C2 - mc) + bs
            m_ref[...] = m_new

        if C % BCC != 0:
            @pl.when(j < NCB - 1)
            def _fast():
                _block(False)

            @pl.when(j == NCB - 1)
            def _slow():
                _block(True)
        else:
            _block(False)

        @pl.when(j == NCB - 1)
        def _finish():
            m = m_ref[...]
            s = s_ref[...]
            a_x = jnp.sum(p_ref[...], axis=0, keepdims=True)
            q = jnp.exp(-_S * _MARGIN)
            ea = jnp.exp2(a_x * _C2 - m * _C2)
            s_adj = jnp.maximum(s - ea * (1.0 - q), ea * q)
            out_ref[...] = _S * m + jnp.log(s_adj) - _S * (a_x - _MARGIN)

    return body, NCB


def _row_losses(cosT, lab2d, BCC):
    C, B = cosT.shape
    body, NCB = _body_factory(B, C, BCC)
    return pl.pallas_call(
        body,
        grid=(NCB,),
        in_specs=[
            pl.BlockSpec((BCC, B), lambda j: (j, 0)),
            pl.BlockSpec((1, B), lambda j: (0, 0)),
        ],
        out_specs=pl.BlockSpec((1, B), lambda j: (0, 0)),
        out_shape=jax.ShapeDtypeStruct((1, B), jnp.float32),
        scratch_shapes=[
            pltpu.VMEM((1, B), jnp.float32),
            pltpu.VMEM((1, B), jnp.float32),
            pltpu.VMEM((8, B), jnp.float32),
        ],
        compiler_params=pltpu.CompilerParams(
            dimension_semantics=("arbitrary",)
        ),
    )(cosT, lab2d)


def _mean_body(x_ref, out_ref):
    B = x_ref.shape[1]
    out_ref[...] = jnp.full((1, 1), jnp.sum(x_ref[...]) * (1.0 / B),
                            jnp.float32)


def _mean_call(row_losses):
    out = pl.pallas_call(
        _mean_body,
        out_shape=jax.ShapeDtypeStruct((1, 1), jnp.float32),
    )(row_losses)
    return out[0, 0]


@jax.jit
def kernel(cosine, label):
    B, _ = cosine.shape
    lab2d = label.astype(jnp.int32).reshape(1, B)
    rl = _row_losses(cosine.T, lab2d, 1024)
    return _mean_call(rl)


# speedup vs baseline: 3.4756x; 1.1439x over previous
import jax
import jax.numpy as jnp
from jax.experimental import pallas as pl
from jax.experimental.pallas import tpu as pltpu

_S = 64.0
_MARGIN = 0.5
_C2 = _S * 1.4426950408889634


def _body_factory(B, C, BCC):
    NCB = pl.cdiv(C, BCC)
    NSL = BCC // 8

    def body(x_ref, lab_ref, out_ref, m_ref, s_ref, p_ref):
        j = pl.program_id(0)

        @pl.when(j == 0)
        def _init():
            m_ref[...] = jnp.full((1, B), -jnp.inf, jnp.float32)
            s_ref[...] = jnp.zeros((1, B), jnp.float32)
            p_ref[...] = jnp.zeros((8, B), jnp.float32)

        lab = lab_ref[...]
        rows8 = jax.lax.broadcasted_iota(jnp.int32, (8, B), 0)

        def _slice(k, masked):
            xk = x_ref[k * 8:(k + 1) * 8, :]
            if masked:
                gid = j * BCC + k * 8 + rows8
                xk = jnp.where(gid < C, xk, -jnp.inf)
            return xk

        def _block(masked):
            macc = jnp.full((8, B), -jnp.inf, jnp.float32)
            pacc = p_ref[...]
            for k in range(NSL):
                xk = _slice(k, masked)
                macc = jnp.maximum(macc, xk)
                pacc = pacc + jnp.where(rows8 == lab - (j * BCC + k * 8),
                                        xk, 0.0)
            p_ref[...] = pacc
            m_old = m_ref[...]
            m_new = jnp.maximum(m_old, jnp.max(macc, axis=0, keepdims=True))
            mc = m_new * _C2
            sacc = jnp.zeros((8, B), jnp.float32)
            for k in range(NSL):
                sacc = sacc + jnp.exp2(_slice(k, masked) * _C2 - mc)
            bs = jnp.sum(sacc, axis=0, keepdims=True)
            s_ref[...] = s_ref[...] * jnp.exp2(m_old * _C2 - mc) + bs
            m_ref[...] = m_new

        if C % BCC != 0:
            @pl.when(j < NCB - 1)
            def _fast():
                _block(False)

            @pl.when(j == NCB - 1)
            def _slow():
                _block(True)
        else:
            _block(False)

        @pl.when(j == NCB - 1)
        def _finish():
            m = m_ref[...]
            s = s_ref[...]
            a_x = jnp.sum(p_ref[...], axis=0, keepdims=True)
            q = jnp.exp(-_S * _MARGIN)
            ea = jnp.exp2(a_x * _C2 - m * _C2)
            s_adj = jnp.maximum(s - ea * (1.0 - q), ea * q)
            out_ref[...] = _S * m + jnp.log(s_adj) - _S * (a_x - _MARGIN)

    return body, NCB


def _row_losses(cosT, lab2d, BCC):
    C, B = cosT.shape
    body, NCB = _body_factory(B, C, BCC)
    return pl.pallas_call(
        body,
        grid=(NCB,),
        in_specs=[
            pl.BlockSpec((BCC, B), lambda j: (j, 0)),
            pl.BlockSpec((1, B), lambda j: (0, 0)),
        ],
        out_specs=pl.BlockSpec((1, B), lambda j: (0, 0)),
        out_shape=jax.ShapeDtypeStruct((1, B), jnp.float32),
        scratch_shapes=[
            pltpu.VMEM((1, B), jnp.float32),
            pltpu.VMEM((1, B), jnp.float32),
            pltpu.VMEM((8, B), jnp.float32),
        ],
        compiler_params=pltpu.CompilerParams(
            dimension_semantics=("arbitrary",)
        ),
    )(cosT, lab2d)


def _mean_body(x_ref, out_ref):
    B = x_ref.shape[1]
    out_ref[...] = jnp.full((1, 1), jnp.sum(x_ref[...]) * (1.0 / B),
                            jnp.float32)


def _mean_call(row_losses):
    out = pl.pallas_call(
        _mean_body,
        out_shape=jax.ShapeDtypeStruct((1, 1), jnp.float32),
    )(row_losses)
    return out[0, 0]


@jax.jit
def kernel(cosine, label):
    B, _ = cosine.shape
    lab2d = label.astype(jnp.int32).reshape(1, B)
    rl = _row_losses(cosine.T, lab2d, 2048)
    return _mean_call(rl)


# speedup vs baseline: 3.6843x; 1.0601x over previous
import jax
import jax.numpy as jnp
from jax.experimental import pallas as pl
from jax.experimental.pallas import tpu as pltpu

_S = 64.0
_MARGIN = 0.5
_C2 = _S * 1.4426950408889634


def _body_factory(B, C, BCC):
    NCB = pl.cdiv(C, BCC)
    NSL = BCC // 8

    def body(x_ref, lab_ref, out_ref, m_ref, s_ref, p_ref):
        j = pl.program_id(0)

        @pl.when(j == 0)
        def _init():
            m_ref[...] = jnp.full((1, B), -jnp.inf, jnp.float32)
            s_ref[...] = jnp.zeros((1, B), jnp.float32)
            p_ref[...] = jnp.zeros((8, B), jnp.float32)

        lab = lab_ref[...]
        rows8 = jax.lax.broadcasted_iota(jnp.int32, (8, B), 0)

        def _slice(k, masked):
            xk = x_ref[k * 8:(k + 1) * 8, :]
            if masked:
                gid = j * BCC + k * 8 + rows8
                xk = jnp.where(gid < C, xk, -jnp.inf)
            return xk

        def _block(masked):
            macc = jnp.full((8, B), -jnp.inf, jnp.float32)
            pacc = p_ref[...]
            for k in range(NSL):
                xk = _slice(k, masked)
                macc = jnp.maximum(macc, xk)
                pacc = pacc + jnp.where(rows8 == lab - (j * BCC + k * 8),
                                        xk, 0.0)
            p_ref[...] = pacc
            m_old = m_ref[...]
            m_new = jnp.maximum(m_old, jnp.max(macc, axis=0, keepdims=True))
            mc = m_new * _C2
            sacc = jnp.zeros((8, B), jnp.float32)
            for k in range(NSL):
                sacc = sacc + jnp.exp2(_slice(k, masked) * _C2 - mc)
            bs = jnp.sum(sacc, axis=0, keepdims=True)
            s_ref[...] = s_ref[...] * jnp.exp2(m_old * _C2 - mc) + bs
            m_ref[...] = m_new

        if C % BCC != 0:
            @pl.when(j < NCB - 1)
            def _fast():
                _block(False)

            @pl.when(j == NCB - 1)
            def _slow():
                _block(True)
        else:
            _block(False)

        @pl.when(j == NCB - 1)
        def _finish():
            m = m_ref[...]
            s = s_ref[...]
            a_x = jnp.sum(p_ref[...], axis=0, keepdims=True)
            q = jnp.exp(-_S * _MARGIN)
            ea = jnp.exp2(a_x * _C2 - m * _C2)
            s_adj = jnp.maximum(s - ea * (1.0 - q), ea * q)
            out_ref[...] = _S * m + jnp.log(s_adj) - _S * (a_x - _MARGIN)

    return body, NCB


def _row_losses(cosT, lab2d, BCC):
    C, B = cosT.shape
    body, NCB = _body_factory(B, C, BCC)
    return pl.pallas_call(
        body,
        grid=(NCB,),
        in_specs=[
            pl.BlockSpec((BCC, B), lambda j: (j, 0)),
            pl.BlockSpec((1, B), lambda j: (0, 0)),
        ],
        out_specs=pl.BlockSpec((1, B), lambda j: (0, 0)),
        out_shape=jax.ShapeDtypeStruct((1, B), jnp.float32),
        scratch_shapes=[
            pltpu.VMEM((1, B), jnp.float32),
            pltpu.VMEM((1, B), jnp.float32),
            pltpu.VMEM((8, B), jnp.float32),
        ],
        compiler_params=pltpu.CompilerParams(
            dimension_semantics=("arbitrary",)
        ),
    )(cosT, lab2d)


def _mean_body(x_ref, out_ref):
    B = x_ref.shape[1]
    out_ref[...] = jnp.full((1, 1), jnp.sum(x_ref[...]) * (1.0 / B),
                            jnp.float32)


def _mean_call(row_losses):
    out = pl.pallas_call(
        _mean_body,
        out_shape=jax.ShapeDtypeStruct((1, 1), jnp.float32),
    )(row_losses)
    return out[0, 0]


@jax.jit
def kernel(cosine, label):
    B, _ = cosine.shape
    lab2d = label.astype(jnp.int32).reshape(1, B)
    rl = _row_losses(cosine.T, lab2d, 4096)
    return _mean_call(rl)
